# baseline (speedup 1.0000x reference)
import functools

import jax
import jax.numpy as jnp
from jax import lax
from jax.experimental import pallas as pl
from jax.experimental.pallas import tpu as pltpu
from jax.experimental.pallas import tpu_sc as plsc

N_TOK = 100000
D = 128
B = 4096
L = 50
LANES = 16
NC = 2
NS = 16
NW = NC * NS
CHUNKS = B // LANES

_HALF = N_TOK // 2
_ROWS_BLK = 5000
_N_BLKS = _HALF // _ROWS_BLK


def _matvec_body(emb_lo_ref, emb_hi_ref, w_ref, out_ref):
    def row(emb_ref):
        return jax.lax.dot_general(
            w_ref[...], emb_ref[...],
            dimension_numbers=(((1,), (1,)), ((), ())),
            preferred_element_type=jnp.float32)

    def to_u32(v32):
        v16 = jax.lax.convert_element_type(v32, jnp.bfloat16)
        u16 = jax.lax.bitcast_convert_type(v16, jnp.uint16)
        return jax.lax.convert_element_type(u16, jnp.uint32)

    word = to_u32(row(emb_lo_ref)) | (to_u32(row(emb_hi_ref)) << 16)
    out_ref[...] = jax.lax.bitcast_convert_type(word, jnp.int32)[None]


def _matvec(emb, w):
    return pl.pallas_call(
        _matvec_body,
        grid=(_N_BLKS,),
        in_specs=[
            pl.BlockSpec((_ROWS_BLK, D), lambda i: (i, 0)),
            pl.BlockSpec((_ROWS_BLK, D), lambda i: (i + _N_BLKS, 0)),
            pl.BlockSpec((1, D), lambda i: (0, 0)),
        ],
        out_specs=pl.BlockSpec((1, 1, _ROWS_BLK), lambda i: (i, 0, 0)),
        out_shape=jax.ShapeDtypeStruct((_N_BLKS, 1, _ROWS_BLK), jnp.int32),
    )(emb, emb, w)


def _make_sc_kernel():
    mesh = plsc.VectorSubcoreMesh(core_axis_name="c", subcore_axis_name="s")

    @functools.partial(
        pl.kernel,
        out_type=jax.ShapeDtypeStruct((L, LANES), jnp.float32),
        mesh=mesh,
        compiler_params=pltpu.CompilerParams(needs_layout_passes=False),
        scratch_types=[
            pltpu.VMEM((_HALF,), jnp.int32),
            pltpu.VMEM((B,), jnp.int32),
            pltpu.VMEM((B,), jnp.float32),
            pltpu.VMEM((B,), jnp.int32),
            pltpu.VMEM((B,), jnp.float32),
            pltpu.VMEM((LANES,), jnp.float32),
            pltpu.VMEM((LANES,), jnp.float32),
            pltpu.SemaphoreType.DMA,
            pltpu.SemaphoreType.DMA,
        ],
    )
    def sc_fn(xt_hbm, st_hbm, t_hbm, b_hbm, out_hbm,
              t_v, x0_v, s0_v, x1_v, s1_v, b_v, o_v, sem0, sem1):
        wid = lax.axis_index("s") * NC + lax.axis_index("c")
        l1 = wid + NW
        has_slot1 = l1 < L

        group0 = [
            pltpu.async_copy(t_hbm.at[i, 0],
                             t_v.at[pl.ds(i * _ROWS_BLK, _ROWS_BLK)], sem0)
            for i in range(_N_BLKS)
        ]
        group0.append(pltpu.async_copy(b_hbm, b_v, sem0))
        group0.append(pltpu.async_copy(xt_hbm.at[wid], x0_v, sem0))
        group0.append(pltpu.async_copy(st_hbm.at[wid], s0_v, sem0))

        @pl.when(has_slot1)
        def _():
            pltpu.async_copy(xt_hbm.at[l1], x1_v, sem1)
            pltpu.async_copy(st_hbm.at[l1], s1_v, sem1)

        for c in group0:
            c.wait()
        bias = b_v[...][0]

        def do_position(l, x_v, s_v):
            unroll = 4

            def chunk(c, acc):
                for u in range(unroll):
                    sl = pl.ds((c * unroll + u) * LANES, LANES)
                    xi = x_v[sl]
                    hi = xi >= _HALF
                    wrd = plsc.load_gather(t_v, [jnp.where(hi, xi - _HALF, xi)])
                    bits = jnp.where(hi, wrd & jnp.int32(-65536), wrd << 16)
                    tv = plsc.bitcast(bits, jnp.float32)
                    acc = acc + tv * s_v[sl]
                return acc

            acc = lax.fori_loop(0, CHUNKS // unroll, chunk,
                                jnp.zeros((LANES,), jnp.float32))
            z = jnp.sum(acc) * (1.0 / B) + bias
            zv = jnp.full((LANES,), z, jnp.float32)
            o_v[...] = 1.0 / (1.0 + jnp.exp(-zv))
            pltpu.sync_copy(o_v, out_hbm.at[l])

        do_position(wid, x0_v, s0_v)

        @pl.when(has_slot1)
        def _():
            pltpu.make_async_copy(xt_hbm.at[l1], x1_v, sem1).wait()
            pltpu.make_async_copy(st_hbm.at[l1], s1_v, sem1).wait()
            do_position(l1, x1_v, s1_v)

    return sc_fn


_sc_fn = _make_sc_kernel()


def kernel(x, score, emb_table, W, b):
    t = _matvec(emb_table, W.astype(jnp.float32))
    xt = x.astype(jnp.int32).T
    st = score[..., 0].astype(jnp.float32).T
    b16 = jnp.broadcast_to(b.astype(jnp.float32), (LANES,))
    out_rows = _sc_fn(xt, st, t, b16)
    return out_rows[:, :1]


# speedup vs baseline: 1.0224x; 1.0224x over previous
import functools

import jax
import jax.numpy as jnp
from jax import lax
from jax.experimental import pallas as pl
from jax.experimental.pallas import tpu as pltpu
from jax.experimental.pallas import tpu_sc as plsc

N_TOK = 100000
D = 128
B = 4096
L = 50
LANES = 16
NC = 2
NS = 16
NW = NC * NS
CHUNKS = B // LANES

_HALF = N_TOK // 2
_ROWS_BLK = 10000
_N_BLKS = _HALF // _ROWS_BLK


def _matvec_body(emb_lo_ref, emb_hi_ref, w_ref, out_ref):
    def row(emb_ref):
        return jax.lax.dot_general(
            w_ref[...], emb_ref[...],
            dimension_numbers=(((1,), (1,)), ((), ())),
            preferred_element_type=jnp.float32)

    def to_u32(v32):
        v16 = jax.lax.convert_element_type(v32, jnp.bfloat16)
        u16 = jax.lax.bitcast_convert_type(v16, jnp.uint16)
        return jax.lax.convert_element_type(u16, jnp.uint32)

    word = to_u32(row(emb_lo_ref)) | (to_u32(row(emb_hi_ref)) << 16)
    out_ref[...] = jax.lax.bitcast_convert_type(word, jnp.int32)[None]


def _matvec(emb, w):
    return pl.pallas_call(
        _matvec_body,
        grid=(_N_BLKS,),
        in_specs=[
            pl.BlockSpec((_ROWS_BLK, D), lambda i: (i, 0)),
            pl.BlockSpec((_ROWS_BLK, D), lambda i: (i + _N_BLKS, 0)),
            pl.BlockSpec((1, D), lambda i: (0, 0)),
        ],
        out_specs=pl.BlockSpec((1, 1, _ROWS_BLK), lambda i: (i, 0, 0)),
        out_shape=jax.ShapeDtypeStruct((_N_BLKS, 1, _ROWS_BLK), jnp.int32),
    )(emb, emb, w)


def _make_sc_kernel():
    mesh = plsc.VectorSubcoreMesh(core_axis_name="c", subcore_axis_name="s")

    @functools.partial(
        pl.kernel,
        out_type=jax.ShapeDtypeStruct((L, LANES), jnp.float32),
        mesh=mesh,
        compiler_params=pltpu.CompilerParams(needs_layout_passes=False),
        scratch_types=[
            pltpu.VMEM((_HALF,), jnp.int32),
            pltpu.VMEM((B,), jnp.int32),
            pltpu.VMEM((B,), jnp.float32),
            pltpu.VMEM((B,), jnp.int32),
            pltpu.VMEM((B,), jnp.float32),
            pltpu.VMEM((LANES,), jnp.float32),
            pltpu.VMEM((LANES,), jnp.float32),
            pltpu.SemaphoreType.DMA,
            pltpu.SemaphoreType.DMA,
        ],
    )
    def sc_fn(xt_hbm, st_hbm, t_hbm, b_hbm, out_hbm,
              t_v, x0_v, s0_v, x1_v, s1_v, b_v, o_v, sem0, sem1):
        wid = lax.axis_index("s") * NC + lax.axis_index("c")
        l1 = wid + NW
        has_slot1 = l1 < L

        group0 = [
            pltpu.async_copy(t_hbm.at[i, 0],
                             t_v.at[pl.ds(i * _ROWS_BLK, _ROWS_BLK)], sem0)
            for i in range(_N_BLKS)
        ]
        group0.append(pltpu.async_copy(b_hbm, b_v, sem0))
        group0.append(pltpu.async_copy(xt_hbm.at[wid], x0_v, sem0))
        group0.append(pltpu.async_copy(st_hbm.at[wid], s0_v, sem0))

        @pl.when(has_slot1)
        def _():
            pltpu.async_copy(xt_hbm.at[l1], x1_v, sem1)
            pltpu.async_copy(st_hbm.at[l1], s1_v, sem1)

        for c in group0:
            c.wait()
        bias = b_v[...][0]

        def do_position(l, x_v, s_v):
            unroll = 4

            def chunk(c, acc):
                for u in range(unroll):
                    sl = pl.ds((c * unroll + u) * LANES, LANES)
                    xi = x_v[sl]
                    hi = xi >= _HALF
                    wrd = plsc.load_gather(t_v, [jnp.where(hi, xi - _HALF, xi)])
                    bits = jnp.where(hi, wrd & jnp.int32(-65536), wrd << 16)
                    tv = plsc.bitcast(bits, jnp.float32)
                    acc = acc + tv * s_v[sl]
                return acc

            acc = lax.fori_loop(0, CHUNKS // unroll, chunk,
                                jnp.zeros((LANES,), jnp.float32))
            z = jnp.sum(acc) * (1.0 / B) + bias
            zv = jnp.full((LANES,), z, jnp.float32)
            o_v[...] = 1.0 / (1.0 + jnp.exp(-zv))
            pltpu.sync_copy(o_v, out_hbm.at[l])

        do_position(wid, x0_v, s0_v)

        @pl.when(has_slot1)
        def _():
            pltpu.make_async_copy(xt_hbm.at[l1], x1_v, sem1).wait()
            pltpu.make_async_copy(st_hbm.at[l1], s1_v, sem1).wait()
            do_position(l1, x1_v, s1_v)

    return sc_fn


_sc_fn = _make_sc_kernel()


def kernel(x, score, emb_table, W, b):
    t = _matvec(emb_table, W.astype(jnp.float32))
    xt = x.astype(jnp.int32).T
    st = score[..., 0].astype(jnp.float32).T
    b16 = jnp.broadcast_to(b.astype(jnp.float32), (LANES,))
    out_rows = _sc_fn(xt, st, t, b16)
    return out_rows[:, :1]
